# baseline (speedup 1.0000x reference)
import functools

import jax
import jax.numpy as jnp
from jax import lax
from jax.experimental import pallas as pl
from jax.experimental.pallas import tpu as pltpu
from jax.experimental.pallas import tpu_sc as plsc

_BETA = 0.25
_BN = 512


def _tc_body(x_ref, e_ref, kio_ref, inds_ref, loss_ref, *, kk):
    xt = x_ref[...].T
    e = e_ref[...]
    xn = jnp.sum(xt * xt, axis=0, keepdims=True)
    en = jnp.sum(e * e, axis=1, keepdims=True)
    xe2 = lax.dot_general(e + e, xt, (((1,), (0,)), ((), ())),
                          preferred_element_type=jnp.float32)
    dist = (xn + en) - xe2
    m = jnp.min(dist, axis=0, keepdims=True)
    idxf = jnp.min(jnp.where(dist == m, kio_ref[...], float(kk)),
                   axis=0, keepdims=True)
    inds_ref[...] = idxf.astype(jnp.int32).reshape(1, 1, -1)
    loss_ref[...] = ((1.0 + _BETA) * m).reshape(1, 1, -1)


def _tc_dist_argmin(latents, embedding):
    n, d = latents.shape
    kk = embedding.shape[0]
    grid = n // _BN
    kio = jnp.arange(kk, dtype=jnp.float32)[:, None]
    inds3, loss3 = pl.pallas_call(
        functools.partial(_tc_body, kk=kk),
        grid=(grid,),
        in_specs=[
            pl.BlockSpec((_BN, d), lambda i: (i, 0)),
            pl.BlockSpec((kk, d), lambda i: (0, 0)),
            pl.BlockSpec((kk, 1), lambda i: (0, 0)),
        ],
        out_specs=[
            pl.BlockSpec((1, 1, _BN), lambda i: (i, 0, 0)),
            pl.BlockSpec((1, 1, _BN), lambda i: (i, 0, 0)),
        ],
        out_shape=[
            jax.ShapeDtypeStruct((grid, 1, _BN), jnp.int32),
            jax.ShapeDtypeStruct((grid, 1, _BN), jnp.float32),
        ],
    )(latents, embedding, kio)
    return inds3.reshape(n), loss3.reshape(n)


def _sc_gather(table, idx):
    n = idx.shape[0]
    dp = table.shape[1]
    info = plsc.get_sparse_core_info()
    nc, ns = info.num_cores, info.num_subcores
    nw = nc * ns
    b_per_w = n // nw
    nchunk = 4
    chunk = b_per_w // nchunk
    mesh = plsc.VectorSubcoreMesh(core_axis_name="c", subcore_axis_name="s")

    @functools.partial(
        pl.kernel,
        out_type=jax.ShapeDtypeStruct((n, dp), jnp.float32),
        mesh=mesh,
        scratch_types=[
            pltpu.VMEM((b_per_w,), jnp.int32),
            pltpu.VMEM((chunk, dp), jnp.float32),
            pltpu.VMEM((chunk, dp), jnp.float32),
            pltpu.SemaphoreType.DMA,
            pltpu.SemaphoreType.DMA,
        ],
    )
    def gather(table_hbm, idx_hbm, out_hbm, idx_v, rows_v0, rows_v1, s0, s1):
        wid = lax.axis_index("s") * nc + lax.axis_index("c")
        base = wid * b_per_w
        rows = (rows_v0, rows_v1)
        sems = (s0, s1)
        pltpu.sync_copy(idx_hbm.at[pl.ds(base, b_per_w)], idx_v)
        prev = pltpu.async_copy(table_hbm.at[idx_v.at[pl.ds(0, chunk)]],
                                rows[0], sems[0])
        for c in range(1, nchunk):
            cur = pltpu.async_copy(
                table_hbm.at[idx_v.at[pl.ds(c * chunk, chunk)]],
                rows[c % 2], sems[c % 2])
            prev.wait()
            pltpu.sync_copy(rows[(c - 1) % 2],
                            out_hbm.at[pl.ds(base + (c - 1) * chunk, chunk)])
            prev = cur
        prev.wait()
        pltpu.sync_copy(rows[(nchunk - 1) % 2],
                        out_hbm.at[pl.ds(base + (nchunk - 1) * chunk, chunk)])

    return gather(table, idx)


def kernel(latents, embedding):
    inds, vq_loss = _tc_dist_argmin(latents, embedding)
    d = embedding.shape[1]
    table = jnp.pad(embedding, ((0, 0), (0, 128 - d)))
    quantized = _sc_gather(table, inds)[:, :d]
    return quantized, vq_loss


# speedup vs baseline: 1.1519x; 1.1519x over previous
import functools

import jax
import jax.numpy as jnp
from jax import lax
from jax.experimental import pallas as pl
from jax.experimental.pallas import tpu as pltpu
from jax.experimental.pallas import tpu_sc as plsc

_BETA = 0.25
_BN = 512


def _tc_body(xt_ref, e_ref, kio_ref, inds_ref, loss_ref, *, kk):
    xt = xt_ref[...]
    e = e_ref[...]
    xn = jnp.sum(xt * xt, axis=0, keepdims=True)
    en = jnp.sum(e * e, axis=1, keepdims=True)
    xe2 = lax.dot_general(e + e, xt, (((1,), (0,)), ((), ())),
                          preferred_element_type=jnp.float32)
    dist = (xn + en) - xe2
    m = jnp.min(dist, axis=0, keepdims=True)
    idxf = jnp.min(jnp.where(dist == m, kio_ref[...], float(kk)),
                   axis=0, keepdims=True)
    inds_ref[...] = idxf.astype(jnp.int32).reshape(1, 1, -1)
    loss_ref[...] = ((1.0 + _BETA) * m).reshape(1, 1, -1)


def _tc_dist_argmin(latents, embedding):
    n, d = latents.shape
    kk = embedding.shape[0]
    grid = n // _BN
    xt = latents.T
    kio = jnp.arange(kk, dtype=jnp.float32)[:, None]
    inds3, loss3 = pl.pallas_call(
        functools.partial(_tc_body, kk=kk),
        grid=(grid,),
        in_specs=[
            pl.BlockSpec((d, _BN), lambda i: (0, i)),
            pl.BlockSpec((kk, d), lambda i: (0, 0)),
            pl.BlockSpec((kk, 1), lambda i: (0, 0)),
        ],
        out_specs=[
            pl.BlockSpec((1, 1, _BN), lambda i: (i, 0, 0)),
            pl.BlockSpec((1, 1, _BN), lambda i: (i, 0, 0)),
        ],
        out_shape=[
            jax.ShapeDtypeStruct((grid, 1, _BN), jnp.int32),
            jax.ShapeDtypeStruct((grid, 1, _BN), jnp.float32),
        ],
    )(xt, embedding, kio)
    return inds3.reshape(n), loss3.reshape(n)


def _sc_gather(table, idx):
    n = idx.shape[0]
    dp = table.shape[1]
    info = plsc.get_sparse_core_info()
    nc, ns = info.num_cores, info.num_subcores
    nw = nc * ns
    b_per_w = n // nw
    nchunk = 4
    chunk = b_per_w // nchunk
    mesh = plsc.VectorSubcoreMesh(core_axis_name="c", subcore_axis_name="s")

    @functools.partial(
        pl.kernel,
        out_type=jax.ShapeDtypeStruct((n, dp), jnp.float32),
        mesh=mesh,
        scratch_types=[
            pltpu.VMEM((b_per_w,), jnp.int32),
            pltpu.VMEM((chunk, dp), jnp.float32),
            pltpu.VMEM((chunk, dp), jnp.float32),
            pltpu.SemaphoreType.DMA,
            pltpu.SemaphoreType.DMA,
        ],
    )
    def gather(table_hbm, idx_hbm, out_hbm, idx_v, rows_v0, rows_v1, s0, s1):
        wid = lax.axis_index("s") * nc + lax.axis_index("c")
        base = wid * b_per_w
        rows = (rows_v0, rows_v1)
        sems = (s0, s1)
        pltpu.sync_copy(idx_hbm.at[pl.ds(base, b_per_w)], idx_v)
        prev = pltpu.async_copy(table_hbm.at[idx_v.at[pl.ds(0, chunk)]],
                                rows[0], sems[0])
        for c in range(1, nchunk):
            cur = pltpu.async_copy(
                table_hbm.at[idx_v.at[pl.ds(c * chunk, chunk)]],
                rows[c % 2], sems[c % 2])
            prev.wait()
            pltpu.sync_copy(rows[(c - 1) % 2],
                            out_hbm.at[pl.ds(base + (c - 1) * chunk, chunk)])
            prev = cur
        prev.wait()
        pltpu.sync_copy(rows[(nchunk - 1) % 2],
                        out_hbm.at[pl.ds(base + (nchunk - 1) * chunk, chunk)])

    return gather(table, idx)


def kernel(latents, embedding):
    inds, vq_loss = _tc_dist_argmin(latents, embedding)
    d = embedding.shape[1]
    table = jnp.pad(embedding, ((0, 0), (0, 128 - d)))
    quantized = _sc_gather(table, inds)[:, :d]
    return quantized, vq_loss


# speedup vs baseline: 1.2991x; 1.1278x over previous
import functools

import jax
import jax.numpy as jnp
from jax import lax
from jax.experimental import pallas as pl
from jax.experimental.pallas import tpu as pltpu
from jax.experimental.pallas import tpu_sc as plsc

_BETA = 0.25
_BN = 1024


def _tc_body(xt_ref, e_ref, kio_ref, inds_ref, loss_ref, *, kk):
    xt = xt_ref[...]
    e = e_ref[...]
    xn = jnp.sum(xt * xt, axis=0, keepdims=True)
    en = jnp.sum(e * e, axis=1, keepdims=True)
    xe2 = lax.dot_general(e + e, xt, (((1,), (0,)), ((), ())),
                          preferred_element_type=jnp.float32)
    dist = (xn + en) - xe2
    m = jnp.min(dist, axis=0, keepdims=True)
    idxf = jnp.min(jnp.where(dist == m, kio_ref[...], float(kk)),
                   axis=0, keepdims=True)
    inds_ref[...] = idxf.astype(jnp.int32).reshape(1, 1, -1)
    loss_ref[...] = ((1.0 + _BETA) * m).reshape(1, 1, -1)


def _tc_dist_argmin(latents, embedding):
    n, d = latents.shape
    kk = embedding.shape[0]
    grid = n // _BN
    xt = latents.T
    kio = jnp.arange(kk, dtype=jnp.float32)[:, None]
    inds3, loss3 = pl.pallas_call(
        functools.partial(_tc_body, kk=kk),
        grid=(grid,),
        in_specs=[
            pl.BlockSpec((d, _BN), lambda i: (0, i)),
            pl.BlockSpec((kk, d), lambda i: (0, 0)),
            pl.BlockSpec((kk, 1), lambda i: (0, 0)),
        ],
        out_specs=[
            pl.BlockSpec((1, 1, _BN), lambda i: (i, 0, 0)),
            pl.BlockSpec((1, 1, _BN), lambda i: (i, 0, 0)),
        ],
        out_shape=[
            jax.ShapeDtypeStruct((grid, 1, _BN), jnp.int32),
            jax.ShapeDtypeStruct((grid, 1, _BN), jnp.float32),
        ],
    )(xt, embedding, kio)
    return inds3.reshape(n), loss3.reshape(n)


def _sc_gather(table, idx):
    n = idx.shape[0]
    dp = table.shape[1]
    info = plsc.get_sparse_core_info()
    nc, ns = info.num_cores, info.num_subcores
    nw = nc * ns
    b_per_w = n // nw
    chunk = b_per_w // 2
    mesh = plsc.VectorSubcoreMesh(core_axis_name="c", subcore_axis_name="s")

    @functools.partial(
        pl.kernel,
        out_type=jax.ShapeDtypeStruct((n, dp), jnp.float32),
        mesh=mesh,
        scratch_types=[
            pltpu.VMEM((b_per_w,), jnp.int32),
            pltpu.VMEM((chunk, dp), jnp.float32),
            pltpu.SemaphoreType.DMA,
        ],
    )
    def gather(table_hbm, idx_hbm, out_hbm, idx_v, rows_v, sem):
        wid = lax.axis_index("s") * nc + lax.axis_index("c")
        base = wid * b_per_w
        pltpu.sync_copy(idx_hbm.at[pl.ds(base, b_per_w)], idx_v)
        for c in range(2):
            pltpu.async_copy(table_hbm.at[idx_v.at[pl.ds(c * chunk, chunk)]],
                             rows_v, sem).wait()
            pltpu.sync_copy(rows_v, out_hbm.at[pl.ds(base + c * chunk, chunk)])

    return gather(table, idx)


def kernel(latents, embedding):
    inds, vq_loss = _tc_dist_argmin(latents, embedding)
    d = embedding.shape[1]
    table = jnp.pad(embedding, ((0, 0), (0, 128 - d)))
    quantized = _sc_gather(table, inds)[:, :d]
    return quantized, vq_loss


# speedup vs baseline: 1.3816x; 1.0635x over previous
import functools

import jax
import jax.numpy as jnp
from jax import lax
from jax.experimental import pallas as pl
from jax.experimental.pallas import tpu as pltpu
from jax.experimental.pallas import tpu_sc as plsc

_BETA = 0.25
_BN = 2048


def _tc_body(xt_ref, e_ref, kio_ref, inds_ref, loss_ref, *, kk):
    xt = xt_ref[...]
    e = e_ref[...]
    xn = jnp.sum(xt * xt, axis=0, keepdims=True)
    en = jnp.sum(e * e, axis=1, keepdims=True)
    xe2 = lax.dot_general(e + e, xt, (((1,), (0,)), ((), ())),
                          preferred_element_type=jnp.float32)
    dist = (xn + en) - xe2
    m = jnp.min(dist, axis=0, keepdims=True)
    idxf = jnp.min(jnp.where(dist == m, kio_ref[...], float(kk)),
                   axis=0, keepdims=True)
    inds_ref[...] = idxf.astype(jnp.int32).reshape(1, 1, -1)
    loss_ref[...] = ((1.0 + _BETA) * m).reshape(1, 1, -1)


def _tc_dist_argmin(latents, embedding):
    n, d = latents.shape
    kk = embedding.shape[0]
    grid = n // _BN
    xt = latents.T
    kio = jnp.arange(kk, dtype=jnp.float32)[:, None]
    inds3, loss3 = pl.pallas_call(
        functools.partial(_tc_body, kk=kk),
        grid=(grid,),
        in_specs=[
            pl.BlockSpec((d, _BN), lambda i: (0, i)),
            pl.BlockSpec((kk, d), lambda i: (0, 0)),
            pl.BlockSpec((kk, 1), lambda i: (0, 0)),
        ],
        out_specs=[
            pl.BlockSpec((1, 1, _BN), lambda i: (i, 0, 0)),
            pl.BlockSpec((1, 1, _BN), lambda i: (i, 0, 0)),
        ],
        out_shape=[
            jax.ShapeDtypeStruct((grid, 1, _BN), jnp.int32),
            jax.ShapeDtypeStruct((grid, 1, _BN), jnp.float32),
        ],
    )(xt, embedding, kio)
    return inds3.reshape(n), loss3.reshape(n)


def _sc_gather(table, idx):
    n = idx.shape[0]
    dp = table.shape[1]
    info = plsc.get_sparse_core_info()
    nc, ns = info.num_cores, info.num_subcores
    nw = nc * ns
    b_per_w = n // nw
    chunk = b_per_w // 2
    mesh = plsc.VectorSubcoreMesh(core_axis_name="c", subcore_axis_name="s")

    @functools.partial(
        pl.kernel,
        out_type=jax.ShapeDtypeStruct((n, dp), jnp.float32),
        mesh=mesh,
        scratch_types=[
            pltpu.VMEM((b_per_w,), jnp.int32),
            pltpu.VMEM((chunk, dp), jnp.float32),
            pltpu.SemaphoreType.DMA,
        ],
    )
    def gather(table_hbm, idx_hbm, out_hbm, idx_v, rows_v, sem):
        wid = lax.axis_index("s") * nc + lax.axis_index("c")
        base = wid * b_per_w
        pltpu.sync_copy(idx_hbm.at[pl.ds(base, b_per_w)], idx_v)
        for c in range(2):
            pltpu.async_copy(table_hbm.at[idx_v.at[pl.ds(c * chunk, chunk)]],
                             rows_v, sem).wait()
            pltpu.sync_copy(rows_v, out_hbm.at[pl.ds(base + c * chunk, chunk)])

    return gather(table, idx)


def kernel(latents, embedding):
    inds, vq_loss = _tc_dist_argmin(latents, embedding)
    d = embedding.shape[1]
    table = jnp.pad(embedding, ((0, 0), (0, 128 - d)))
    quantized = _sc_gather(table, inds)[:, :d]
    return quantized, vq_loss


# speedup vs baseline: 1.4042x; 1.0164x over previous
import functools

import jax
import jax.numpy as jnp
from jax import lax
from jax.experimental import pallas as pl
from jax.experimental.pallas import tpu as pltpu
from jax.experimental.pallas import tpu_sc as plsc

_BETA = 0.25
_BN = 4096


def _tc_body(xt_ref, e_ref, kio_ref, inds_ref, loss_ref, *, kk):
    xt = xt_ref[...]
    e = e_ref[...]
    xn = jnp.sum(xt * xt, axis=0, keepdims=True)
    en = jnp.sum(e * e, axis=1, keepdims=True)
    xe2 = lax.dot_general(e + e, xt, (((1,), (0,)), ((), ())),
                          preferred_element_type=jnp.float32)
    dist = (xn + en) - xe2
    m = jnp.min(dist, axis=0, keepdims=True)
    idxf = jnp.min(jnp.where(dist == m, kio_ref[...], float(kk)),
                   axis=0, keepdims=True)
    inds_ref[...] = idxf.astype(jnp.int32).reshape(1, 1, -1)
    loss_ref[...] = ((1.0 + _BETA) * m).reshape(1, 1, -1)


def _tc_dist_argmin(latents, embedding):
    n, d = latents.shape
    kk = embedding.shape[0]
    grid = n // _BN
    xt = latents.T
    kio = jnp.arange(kk, dtype=jnp.float32)[:, None]
    inds3, loss3 = pl.pallas_call(
        functools.partial(_tc_body, kk=kk),
        grid=(grid,),
        in_specs=[
            pl.BlockSpec((d, _BN), lambda i: (0, i)),
            pl.BlockSpec((kk, d), lambda i: (0, 0)),
            pl.BlockSpec((kk, 1), lambda i: (0, 0)),
        ],
        out_specs=[
            pl.BlockSpec((1, 1, _BN), lambda i: (i, 0, 0)),
            pl.BlockSpec((1, 1, _BN), lambda i: (i, 0, 0)),
        ],
        out_shape=[
            jax.ShapeDtypeStruct((grid, 1, _BN), jnp.int32),
            jax.ShapeDtypeStruct((grid, 1, _BN), jnp.float32),
        ],
    )(xt, embedding, kio)
    return inds3.reshape(n), loss3.reshape(n)


def _sc_gather(table, idx):
    n = idx.shape[0]
    dp = table.shape[1]
    info = plsc.get_sparse_core_info()
    nc, ns = info.num_cores, info.num_subcores
    nw = nc * ns
    b_per_w = n // nw
    chunk = b_per_w // 2
    mesh = plsc.VectorSubcoreMesh(core_axis_name="c", subcore_axis_name="s")

    @functools.partial(
        pl.kernel,
        out_type=jax.ShapeDtypeStruct((n, dp), jnp.float32),
        mesh=mesh,
        scratch_types=[
            pltpu.VMEM((b_per_w,), jnp.int32),
            pltpu.VMEM((chunk, dp), jnp.float32),
            pltpu.SemaphoreType.DMA,
        ],
    )
    def gather(table_hbm, idx_hbm, out_hbm, idx_v, rows_v, sem):
        wid = lax.axis_index("s") * nc + lax.axis_index("c")
        base = wid * b_per_w
        pltpu.sync_copy(idx_hbm.at[pl.ds(base, b_per_w)], idx_v)
        for c in range(2):
            pltpu.async_copy(table_hbm.at[idx_v.at[pl.ds(c * chunk, chunk)]],
                             rows_v, sem).wait()
            pltpu.sync_copy(rows_v, out_hbm.at[pl.ds(base + c * chunk, chunk)])

    return gather(table, idx)


def kernel(latents, embedding):
    inds, vq_loss = _tc_dist_argmin(latents, embedding)
    d = embedding.shape[1]
    table = jnp.pad(embedding, ((0, 0), (0, 128 - d)))
    quantized = _sc_gather(table, inds)[:, :d]
    return quantized, vq_loss


# speedup vs baseline: 1.4304x; 1.0187x over previous
import functools

import jax
import jax.numpy as jnp
from jax import lax
from jax.experimental import pallas as pl
from jax.experimental.pallas import tpu as pltpu
from jax.experimental.pallas import tpu_sc as plsc

_BETA = 0.25
_BN = 8192


def _tc_body(xt_ref, e_ref, kio_ref, inds_ref, loss_ref, *, kk):
    xt = xt_ref[...]
    e = e_ref[...]
    xn = jnp.sum(xt * xt, axis=0, keepdims=True)
    en = jnp.sum(e * e, axis=1, keepdims=True)
    xe2 = lax.dot_general(e + e, xt, (((1,), (0,)), ((), ())),
                          preferred_element_type=jnp.float32)
    dist = (xn + en) - xe2
    m = jnp.min(dist, axis=0, keepdims=True)
    idxf = jnp.min(jnp.where(dist == m, kio_ref[...], float(kk)),
                   axis=0, keepdims=True)
    inds_ref[...] = idxf.astype(jnp.int32).reshape(1, 1, -1)
    loss_ref[...] = ((1.0 + _BETA) * m).reshape(1, 1, -1)


def _tc_dist_argmin(latents, embedding):
    n, d = latents.shape
    kk = embedding.shape[0]
    grid = n // _BN
    xt = latents.T
    kio = jnp.arange(kk, dtype=jnp.float32)[:, None]
    inds3, loss3 = pl.pallas_call(
        functools.partial(_tc_body, kk=kk),
        grid=(grid,),
        in_specs=[
            pl.BlockSpec((d, _BN), lambda i: (0, i)),
            pl.BlockSpec((kk, d), lambda i: (0, 0)),
            pl.BlockSpec((kk, 1), lambda i: (0, 0)),
        ],
        out_specs=[
            pl.BlockSpec((1, 1, _BN), lambda i: (i, 0, 0)),
            pl.BlockSpec((1, 1, _BN), lambda i: (i, 0, 0)),
        ],
        out_shape=[
            jax.ShapeDtypeStruct((grid, 1, _BN), jnp.int32),
            jax.ShapeDtypeStruct((grid, 1, _BN), jnp.float32),
        ],
    )(xt, embedding, kio)
    return inds3.reshape(n), loss3.reshape(n)


def _sc_gather(table, idx):
    n = idx.shape[0]
    dp = table.shape[1]
    info = plsc.get_sparse_core_info()
    nc, ns = info.num_cores, info.num_subcores
    nw = nc * ns
    b_per_w = n // nw
    chunk = b_per_w // 2
    mesh = plsc.VectorSubcoreMesh(core_axis_name="c", subcore_axis_name="s")

    @functools.partial(
        pl.kernel,
        out_type=jax.ShapeDtypeStruct((n, dp), jnp.float32),
        mesh=mesh,
        scratch_types=[
            pltpu.VMEM((b_per_w,), jnp.int32),
            pltpu.VMEM((chunk, dp), jnp.float32),
            pltpu.SemaphoreType.DMA,
        ],
    )
    def gather(table_hbm, idx_hbm, out_hbm, idx_v, rows_v, sem):
        wid = lax.axis_index("s") * nc + lax.axis_index("c")
        base = wid * b_per_w
        pltpu.sync_copy(idx_hbm.at[pl.ds(base, b_per_w)], idx_v)
        for c in range(2):
            pltpu.async_copy(table_hbm.at[idx_v.at[pl.ds(c * chunk, chunk)]],
                             rows_v, sem).wait()
            pltpu.sync_copy(rows_v, out_hbm.at[pl.ds(base + c * chunk, chunk)])

    return gather(table, idx)


def kernel(latents, embedding):
    inds, vq_loss = _tc_dist_argmin(latents, embedding)
    d = embedding.shape[1]
    table = jnp.pad(embedding, ((0, 0), (0, 128 - d)))
    quantized = _sc_gather(table, inds)[:, :d]
    return quantized, vq_loss
